# Initial kernel scaffold; baseline (speedup 1.0000x reference)
import jax
import jax.numpy as jnp
from jax.experimental import pallas as pl


def kernel(vertex_attr, edgeij_pair, edge_attr, g, batch):
    raise NotImplementedError("write your pallas kernel here")


# speedup vs baseline: 46.1995x; 46.1995x over previous
import functools

import jax
import jax.numpy as jnp
from jax import lax
from jax.experimental import pallas as pl
from jax.experimental.pallas import tpu as pltpu
from jax.experimental.pallas import tpu_sc as plsc

V = 100000
E = 6400000
NC = 2
NS = 16
NW = NC * NS
VPAD = 100352
EPW = E // NW
CHUNK = 10000
NCHUNK = EPW // CHUNK
SL = VPAD // NS
VB = VPAD // NW


def _scatter_body(dst_hbm, val_hbm, zeros_hbm, partial_hbm, idx_v, val_v, acc):
    cid = lax.axis_index("c")
    sid = lax.axis_index("s")
    wid = cid * NS + sid

    pltpu.sync_copy(zeros_hbm.at[pl.ds(sid * SL, SL)], acc.at[pl.ds(sid * SL, SL)])
    plsc.subcore_barrier()

    @pl.loop(0, NCHUNK)
    def _chunks(k):
        off = pl.multiple_of(wid * EPW + k * CHUNK, 8)
        pltpu.sync_copy(dst_hbm.at[pl.ds(off, CHUNK)], idx_v)
        pltpu.sync_copy(val_hbm.at[pl.ds(off, CHUNK)], val_v)
        pltpu.sync_copy(val_v, acc.at[idx_v], add=True)

    plsc.subcore_barrier()
    pltpu.sync_copy(acc.at[pl.ds(sid * SL, SL)],
                    partial_hbm.at[pl.ds(cid * VPAD + sid * SL, SL)])


def _finalize_body(partial_hbm, b_hbm, r_hbm, p0_v, p1_v, b_v, r_v):
    cid = lax.axis_index("c")
    sid = lax.axis_index("s")
    wid = cid * NS + sid
    base = wid * VB

    pltpu.sync_copy(partial_hbm.at[pl.ds(base, VB)], p0_v)
    pltpu.sync_copy(partial_hbm.at[pl.ds(VPAD + base, VB)], p1_v)
    pltpu.sync_copy(b_hbm.at[pl.ds(base, VB)], b_v)

    @pl.loop(0, VB // 16)
    def _rows(i):
        s = pl.ds(i * 16, 16)
        r_v[s] = b_v[s] - (p0_v[s] + p1_v[s])

    pltpu.sync_copy(r_v, r_hbm.at[pl.ds(base, VB)])


def kernel(vertex_attr, edgeij_pair, edge_attr, g, batch):
    del g, batch
    mesh = plsc.VectorSubcoreMesh(core_axis_name="c", subcore_axis_name="s")

    scatter_k = pl.kernel(
        _scatter_body,
        out_type=jax.ShapeDtypeStruct((NC * VPAD,), jnp.float32),
        mesh=mesh,
        scratch_types=[
            pltpu.VMEM((CHUNK,), jnp.int32),
            pltpu.VMEM((CHUNK,), jnp.float32),
            pltpu.VMEM_SHARED((VPAD,), jnp.float32),
        ],
    )
    finalize_k = pl.kernel(
        _finalize_body,
        out_type=jax.ShapeDtypeStruct((VPAD,), jnp.float32),
        mesh=mesh,
        scratch_types=[
            pltpu.VMEM((VB,), jnp.float32),
            pltpu.VMEM((VB,), jnp.float32),
            pltpu.VMEM((VB,), jnp.float32),
            pltpu.VMEM((VB,), jnp.float32),
        ],
    )

    dst = edgeij_pair[1]
    zeros = jnp.zeros((VPAD,), jnp.float32)
    partial = scatter_k(dst, edge_attr, zeros)
    b_col = vertex_attr[:, 0]
    x_col = vertex_attr[:, 1]
    b_pad = jnp.pad(b_col, (0, VPAD - V))
    r = finalize_k(partial, b_pad)[:V]
    return jnp.stack([b_col, x_col, r], axis=1)


# speedup vs baseline: 55.0074x; 1.1907x over previous
import functools

import jax
import jax.numpy as jnp
from jax import lax
from jax.experimental import pallas as pl
from jax.experimental.pallas import tpu as pltpu
from jax.experimental.pallas import tpu_sc as plsc

V = 100000
E = 6400000
NC = 2
NS = 16
NW = NC * NS
VPAD = 100352
EPW = E // NW
CHUNK = 10000
NCHUNK = EPW // CHUNK
SL = VPAD // NS
VB = VPAD // NW


def _scatter_body(eij_hbm, val_hbm, zeros_hbm, partial_hbm,
                  idx_v0, idx_v1, val_v0, val_v1, acc, sem0, sem1):
    cid = lax.axis_index("c")
    sid = lax.axis_index("s")
    wid = cid * NS + sid
    base = wid * EPW
    idx_bufs = (idx_v0, idx_v1)
    val_bufs = (val_v0, val_v1)
    sems = (sem0, sem1)

    pltpu.sync_copy(zeros_hbm.at[pl.ds(sid * SL, SL)], acc.at[pl.ds(sid * SL, SL)])
    plsc.subcore_barrier()

    def _start(k, b):
        off = pl.multiple_of(base + k * CHUNK, 8)
        d1 = pltpu.async_copy(eij_hbm.at[pl.ds(E + off, CHUNK)], idx_bufs[b], sems[b])
        d2 = pltpu.async_copy(val_hbm.at[pl.ds(off, CHUNK)], val_bufs[b], sems[b])
        return d1, d2

    descs = [None, None]
    descs[0] = _start(0, 0)
    for k in range(NCHUNK):
        b = k % 2
        if k + 1 < NCHUNK:
            descs[1 - b] = _start(k + 1, 1 - b)
        d1, d2 = descs[b]
        d1.wait()
        d2.wait()
        pltpu.sync_copy(val_bufs[b], acc.at[idx_bufs[b]], add=True)

    plsc.subcore_barrier()
    pltpu.sync_copy(acc.at[pl.ds(sid * SL, SL)],
                    partial_hbm.at[pl.ds(cid * VPAD + sid * SL, SL)])


def _finalize_body(partial_hbm, b_hbm, r_hbm, p0_v, p1_v, b_v, r_v):
    cid = lax.axis_index("c")
    sid = lax.axis_index("s")
    wid = cid * NS + sid
    base = wid * VB

    pltpu.sync_copy(partial_hbm.at[pl.ds(base, VB)], p0_v)
    pltpu.sync_copy(partial_hbm.at[pl.ds(VPAD + base, VB)], p1_v)
    pltpu.sync_copy(b_hbm.at[pl.ds(base, VB)], b_v)

    @pl.loop(0, VB // 16)
    def _rows(i):
        s = pl.ds(i * 16, 16)
        r_v[s] = b_v[s] - (p0_v[s] + p1_v[s])

    pltpu.sync_copy(r_v, r_hbm.at[pl.ds(base, VB)])


def kernel(vertex_attr, edgeij_pair, edge_attr, g, batch):
    del g, batch
    mesh = plsc.VectorSubcoreMesh(core_axis_name="c", subcore_axis_name="s")

    scatter_k = pl.kernel(
        _scatter_body,
        out_type=jax.ShapeDtypeStruct((NC * VPAD,), jnp.float32),
        mesh=mesh,
        scratch_types=[
            pltpu.VMEM((CHUNK,), jnp.int32),
            pltpu.VMEM((CHUNK,), jnp.int32),
            pltpu.VMEM((CHUNK,), jnp.float32),
            pltpu.VMEM((CHUNK,), jnp.float32),
            pltpu.VMEM_SHARED((VPAD,), jnp.float32),
            pltpu.SemaphoreType.DMA,
            pltpu.SemaphoreType.DMA,
        ],
    )
    finalize_k = pl.kernel(
        _finalize_body,
        out_type=jax.ShapeDtypeStruct((VPAD,), jnp.float32),
        mesh=mesh,
        scratch_types=[
            pltpu.VMEM((VB,), jnp.float32),
            pltpu.VMEM((VB,), jnp.float32),
            pltpu.VMEM((VB,), jnp.float32),
            pltpu.VMEM((VB,), jnp.float32),
        ],
    )

    eij_flat = edgeij_pair.reshape(-1)
    zeros = jnp.zeros((VPAD,), jnp.float32)
    partial = scatter_k(eij_flat, edge_attr, zeros)
    b_col = vertex_attr[:, 0]
    x_col = vertex_attr[:, 1]
    b_pad = jnp.pad(b_col, (0, VPAD - V))
    r = finalize_k(partial, b_pad)[:V]
    return jnp.stack([b_col, x_col, r], axis=1)


# speedup vs baseline: 59.0341x; 1.0732x over previous
import functools

import jax
import jax.numpy as jnp
from jax import lax
from jax.experimental import pallas as pl
from jax.experimental.pallas import tpu as pltpu
from jax.experimental.pallas import tpu_sc as plsc

V = 100000
E = 6400000
NC = 2
NS = 16
NW = NC * NS
VPAD = 100352
CHUNK = 12800
NCHUNKS = E // CHUNK
CPW = NCHUNKS // NW
NTAIL = NCHUNKS - CPW * NW
SL = VPAD // NS
VB = VPAD // NW


def _scatter_body(dst_hbm, val_hbm, zeros_hbm, partial_hbm,
                  idx_v0, idx_v1, val_v0, val_v1, acc, sem0, sem1):
    cid = lax.axis_index("c")
    sid = lax.axis_index("s")
    wid = cid * NS + sid
    idx_bufs = (idx_v0, idx_v1)
    val_bufs = (val_v0, val_v1)
    sems = (sem0, sem1)

    pltpu.sync_copy(zeros_hbm.at[pl.ds(sid * SL, SL)], acc.at[pl.ds(sid * SL, SL)])
    plsc.subcore_barrier()

    def _start(chunk_no, b):
        off = pl.multiple_of(chunk_no * CHUNK, 128)
        d1 = pltpu.async_copy(dst_hbm.at[pl.ds(off, CHUNK)], idx_bufs[b], sems[b])
        d2 = pltpu.async_copy(val_hbm.at[pl.ds(off, CHUNK)], val_bufs[b], sems[b])
        return d1, d2

    descs = [None, None]
    descs[0] = _start(wid * CPW, 0)
    for k in range(CPW):
        b = k % 2
        if k + 1 < CPW:
            descs[1 - b] = _start(wid * CPW + k + 1, 1 - b)
        d1, d2 = descs[b]
        d1.wait()
        d2.wait()
        pltpu.sync_copy(val_bufs[b], acc.at[idx_bufs[b]], add=True)

    @pl.when(wid < NTAIL)
    def _tail():
        b = CPW % 2
        d1, d2 = _start(NW * CPW + wid, b)
        d1.wait()
        d2.wait()
        pltpu.sync_copy(val_bufs[b], acc.at[idx_bufs[b]], add=True)

    plsc.subcore_barrier()
    pltpu.sync_copy(acc.at[pl.ds(sid * SL, SL)],
                    partial_hbm.at[pl.ds(cid * VPAD + sid * SL, SL)])


def _finalize_body(partial_hbm, b_hbm, r_hbm, p0_v, p1_v, b_v, r_v):
    cid = lax.axis_index("c")
    sid = lax.axis_index("s")
    wid = cid * NS + sid
    base = wid * VB

    pltpu.sync_copy(partial_hbm.at[pl.ds(base, VB)], p0_v)
    pltpu.sync_copy(partial_hbm.at[pl.ds(VPAD + base, VB)], p1_v)
    pltpu.sync_copy(b_hbm.at[pl.ds(base, VB)], b_v)

    @pl.loop(0, VB // 16)
    def _rows(i):
        s = pl.ds(i * 16, 16)
        r_v[s] = b_v[s] - (p0_v[s] + p1_v[s])

    pltpu.sync_copy(r_v, r_hbm.at[pl.ds(base, VB)])


def kernel(vertex_attr, edgeij_pair, edge_attr, g, batch):
    del g, batch
    mesh = plsc.VectorSubcoreMesh(core_axis_name="c", subcore_axis_name="s")

    scatter_k = pl.kernel(
        _scatter_body,
        out_type=jax.ShapeDtypeStruct((NC * VPAD,), jnp.float32),
        mesh=mesh,
        scratch_types=[
            pltpu.VMEM((CHUNK,), jnp.int32),
            pltpu.VMEM((CHUNK,), jnp.int32),
            pltpu.VMEM((CHUNK,), jnp.float32),
            pltpu.VMEM((CHUNK,), jnp.float32),
            pltpu.VMEM_SHARED((VPAD,), jnp.float32),
            pltpu.SemaphoreType.DMA,
            pltpu.SemaphoreType.DMA,
        ],
    )
    finalize_k = pl.kernel(
        _finalize_body,
        out_type=jax.ShapeDtypeStruct((VPAD,), jnp.float32),
        mesh=mesh,
        scratch_types=[
            pltpu.VMEM((VB,), jnp.float32),
            pltpu.VMEM((VB,), jnp.float32),
            pltpu.VMEM((VB,), jnp.float32),
            pltpu.VMEM((VB,), jnp.float32),
        ],
    )

    zeros = jnp.zeros((VPAD,), jnp.float32)
    partial = scatter_k(edgeij_pair[1], edge_attr, zeros)
    b_col = vertex_attr[:, 0]
    x_col = vertex_attr[:, 1]
    b_pad = jnp.pad(b_col, (0, VPAD - V))
    r = finalize_k(partial, b_pad)[:V]
    return jnp.stack([b_col, x_col, r], axis=1)


# speedup vs baseline: 60.4136x; 1.0234x over previous
import functools

import jax
import jax.numpy as jnp
from jax import lax
from jax.experimental import pallas as pl
from jax.experimental.pallas import tpu as pltpu
from jax.experimental.pallas import tpu_sc as plsc

V = 100000
E = 6400000
NC = 2
NS = 16
NW = NC * NS
VPAD = 100352
CHUNK = 12800
NCHUNKS = E // CHUNK
CPW = NCHUNKS // NW
NTAIL = NCHUNKS - CPW * NW
SL = VPAD // NS
VB = VPAD // NW


def _scatter_body(eij_hbm, val_hbm, zeros_hbm, partial_hbm,
                  eij_v0, eij_v1, idx_v0, idx_v1, val_v0, val_v1, acc,
                  lsem0, lsem1, ssem0, ssem1):
    cid = lax.axis_index("c")
    sid = lax.axis_index("s")
    wid = cid * NS + sid
    eij_bufs = (eij_v0, eij_v1)
    idx_bufs = (idx_v0, idx_v1)
    val_bufs = (val_v0, val_v1)
    lsems = (lsem0, lsem1)
    ssems = (ssem0, ssem1)

    pltpu.sync_copy(zeros_hbm.at[pl.ds(sid * SL, SL)], acc.at[pl.ds(sid * SL, SL)])
    plsc.subcore_barrier()

    def _start(chunk_no, b):
        off = pl.multiple_of(chunk_no * CHUNK, 128)
        d1 = pltpu.async_copy(eij_hbm.at[:, pl.ds(off, CHUNK)], eij_bufs[b], lsems[b])
        d2 = pltpu.async_copy(val_hbm.at[pl.ds(off, CHUNK)], val_bufs[b], lsems[b])
        return d1, d2

    def _extract(b):
        @pl.loop(0, CHUNK // 16, unroll=8)
        def _blk(j):
            s = pl.ds(j * 16, 16)
            idx_bufs[b][s] = eij_bufs[b][1, s]

    ld = [_start(wid * CPW, 0), None]
    sd = [None, None]
    for k in range(CPW):
        b = k % 2
        d1, d2 = ld[b]
        d1.wait()
        d2.wait()
        if sd[b] is not None:
            sd[b].wait()
        _extract(b)
        sd[b] = pltpu.async_copy(val_bufs[b], acc.at[idx_bufs[b]], ssems[b],
                                 add=True)
        if k + 1 < CPW:
            if sd[1 - b] is not None:
                sd[1 - b].wait()
                sd[1 - b] = None
            ld[1 - b] = _start(wid * CPW + k + 1, 1 - b)
    for b in range(2):
        if sd[b] is not None:
            sd[b].wait()

    @pl.when(wid < NTAIL)
    def _tail():
        d1, d2 = _start(NW * CPW + wid, 0)
        d1.wait()
        d2.wait()
        _extract(0)
        pltpu.sync_copy(val_bufs[0], acc.at[idx_bufs[0]], add=True)

    plsc.subcore_barrier()
    pltpu.sync_copy(acc.at[pl.ds(sid * SL, SL)],
                    partial_hbm.at[pl.ds(cid * VPAD + sid * SL, SL)])


def _finalize_body(partial_hbm, b_hbm, r_hbm, p0_v, p1_v, b_v, r_v):
    cid = lax.axis_index("c")
    sid = lax.axis_index("s")
    wid = cid * NS + sid
    base = wid * VB

    pltpu.sync_copy(partial_hbm.at[pl.ds(base, VB)], p0_v)
    pltpu.sync_copy(partial_hbm.at[pl.ds(VPAD + base, VB)], p1_v)
    pltpu.sync_copy(b_hbm.at[pl.ds(base, VB)], b_v)

    @pl.loop(0, VB // 16)
    def _rows(i):
        s = pl.ds(i * 16, 16)
        r_v[s] = b_v[s] - (p0_v[s] + p1_v[s])

    pltpu.sync_copy(r_v, r_hbm.at[pl.ds(base, VB)])


def kernel(vertex_attr, edgeij_pair, edge_attr, g, batch):
    del g, batch
    mesh = plsc.VectorSubcoreMesh(core_axis_name="c", subcore_axis_name="s")

    scatter_k = pl.kernel(
        _scatter_body,
        out_type=jax.ShapeDtypeStruct((NC * VPAD,), jnp.float32),
        mesh=mesh,
        scratch_types=[
            pltpu.VMEM((2, CHUNK), jnp.int32),
            pltpu.VMEM((2, CHUNK), jnp.int32),
            pltpu.VMEM((CHUNK,), jnp.int32),
            pltpu.VMEM((CHUNK,), jnp.int32),
            pltpu.VMEM((CHUNK,), jnp.float32),
            pltpu.VMEM((CHUNK,), jnp.float32),
            pltpu.VMEM_SHARED((VPAD,), jnp.float32),
            pltpu.SemaphoreType.DMA,
            pltpu.SemaphoreType.DMA,
            pltpu.SemaphoreType.DMA,
            pltpu.SemaphoreType.DMA,
        ],
    )
    finalize_k = pl.kernel(
        _finalize_body,
        out_type=jax.ShapeDtypeStruct((VPAD,), jnp.float32),
        mesh=mesh,
        scratch_types=[
            pltpu.VMEM((VB,), jnp.float32),
            pltpu.VMEM((VB,), jnp.float32),
            pltpu.VMEM((VB,), jnp.float32),
            pltpu.VMEM((VB,), jnp.float32),
        ],
    )

    zeros = jnp.zeros((VPAD,), jnp.float32)
    partial = scatter_k(edgeij_pair, edge_attr, zeros)
    b_col = vertex_attr[:, 0]
    x_col = vertex_attr[:, 1]
    b_pad = jnp.pad(b_col, (0, VPAD - V))
    r = finalize_k(partial, b_pad)[:V]
    return jnp.stack([b_col, x_col, r], axis=1)
